# Initial kernel scaffold; baseline (speedup 1.0000x reference)
import jax
import jax.numpy as jnp
from jax.experimental import pallas as pl


def kernel(txt, seg, tok_table, pos_table, seg_table, Wt, bt, Wp, bp, Ws, bs):
    raise NotImplementedError("write your pallas kernel here")


# speedup vs baseline: 15.8909x; 15.8909x over previous
import functools

import jax
import jax.numpy as jnp
from jax import lax
from jax.experimental import pallas as pl
from jax.experimental.pallas import tpu as pltpu
from jax.experimental.pallas import tpu_sc as plsc

B = 1024
L = 200
DK = 128
DM = 768
ROWS = B * L
CHUNK = 128
PAD_ID = 0
RB = 8


def _sc_gather(table, idx_flat, nc, ns):
    nw = nc * ns
    rows_per_w = ROWS // nw
    nchunk = rows_per_w // CHUNK
    mesh = plsc.VectorSubcoreMesh(core_axis_name="c", subcore_axis_name="s")

    @functools.partial(
        pl.kernel,
        mesh=mesh,
        out_type=jax.ShapeDtypeStruct((ROWS, DK), jnp.float32),
        scratch_types=[
            pltpu.VMEM((rows_per_w,), jnp.int32),
            pltpu.VMEM((CHUNK, DK), jnp.float32),
            pltpu.SemaphoreType.DMA,
        ],
    )
    def gather_kernel(table_hbm, idx_hbm, out_hbm, idx_v, rows_v, sem):
        wid = lax.axis_index("s") * nc + lax.axis_index("c")
        pltpu.sync_copy(idx_hbm.at[pl.ds(wid * rows_per_w, rows_per_w)], idx_v)

        def step(j, carry):
            pltpu.async_copy(
                table_hbm.at[idx_v.at[pl.ds(j * CHUNK, CHUNK)]], rows_v, sem
            ).wait()
            pltpu.sync_copy(
                rows_v, out_hbm.at[pl.ds(wid * rows_per_w + j * CHUNK, CHUNK)]
            )
            return carry

        lax.fori_loop(0, nchunk, step, 0)

    return gather_kernel(table, idx_flat)


def _proj_body(pos_ref, wp_ref, segp_ref, ws_ref, bt_ref, bp_ref, bs_ref,
               posproj_ref, segproj_ref):
    bsum = bt_ref[...] + bp_ref[...] + bs_ref[...]
    posproj_ref[...] = (
        jnp.dot(pos_ref[...], wp_ref[...], preferred_element_type=jnp.float32)
        + bsum
    )
    segproj_ref[...] = jnp.dot(
        segp_ref[...], ws_ref[...], preferred_element_type=jnp.float32
    )


def _main_body(gath_ref, txt_ref, seg_ref, wt_ref, posproj_ref, segproj_ref,
               out_ref):
    g = gath_ref[...]
    mask = (txt_ref[...] != PAD_ID).astype(jnp.float32)
    g = g * mask[..., None]
    acc = jnp.dot(
        g.reshape(RB * L, DK), wt_ref[...], preferred_element_type=jnp.float32
    ).reshape(RB, L, DM)
    acc = acc + posproj_ref[...][None]
    s = seg_ref[...]
    sp1 = segproj_ref[1:2, :][None]
    sp2 = segproj_ref[2:3, :][None]
    acc = acc + (s == 1).astype(jnp.float32)[..., None] * sp1
    acc = acc + (s == 2).astype(jnp.float32)[..., None] * sp2
    out_ref[...] = acc


def kernel(txt, seg, tok_table, pos_table, seg_table, Wt, bt, Wp, bp, Ws, bs):
    info = plsc.get_sparse_core_info()
    nc, ns = info.num_cores, info.num_subcores
    nw = nc * ns

    idx_flat = txt.reshape(ROWS)
    gathered = _sc_gather(tok_table, idx_flat, nc, ns)
    gath3 = gathered.reshape(B, L, DK)

    seg_pad = jnp.zeros((8, DK), jnp.float32).at[:3].set(seg_table)
    posproj, segproj = pl.pallas_call(
        _proj_body,
        out_shape=(
            jax.ShapeDtypeStruct((L, DM), jnp.float32),
            jax.ShapeDtypeStruct((8, DM), jnp.float32),
        ),
    )(pos_table, Wp, seg_pad, Ws,
      bt.reshape(1, DM), bp.reshape(1, DM), bs.reshape(1, DM))

    out = pl.pallas_call(
        _main_body,
        grid=(B // RB,),
        in_specs=[
            pl.BlockSpec((RB, L, DK), lambda i: (i, 0, 0)),
            pl.BlockSpec((RB, L), lambda i: (i, 0)),
            pl.BlockSpec((RB, L), lambda i: (i, 0)),
            pl.BlockSpec((DK, DM), lambda i: (0, 0)),
            pl.BlockSpec((L, DM), lambda i: (0, 0)),
            pl.BlockSpec((8, DM), lambda i: (0, 0)),
        ],
        out_specs=pl.BlockSpec((RB, L, DM), lambda i: (i, 0, 0)),
        out_shape=jax.ShapeDtypeStruct((B, L, DM), jnp.float32),
        compiler_params=pltpu.CompilerParams(
            dimension_semantics=("parallel",),
        ),
    )(gath3, txt, seg, Wt, posproj, segproj)
    return out
